# baseline (speedup 1.0000x reference)
import functools

import jax
import jax.numpy as jnp
from jax import lax
from jax.experimental import pallas as pl
from jax.experimental.pallas import tpu as pltpu
from jax.experimental.pallas import tpu_sc as plsc

_BATCH = 1024
_SEQ = 200
_HIDDEN = 128

_info = plsc.get_sparse_core_info()
_NC = _info.num_cores
_NS = _info.num_subcores
_NW = _NC * _NS
_B_PER_W = _BATCH // _NW


def _sc_gather(pert_ids, emb):
    mesh = plsc.VectorSubcoreMesh(core_axis_name="c", subcore_axis_name="s")

    @functools.partial(
        pl.kernel,
        mesh=mesh,
        out_type=jax.ShapeDtypeStruct((_BATCH, _HIDDEN), jnp.float32),
        scratch_types=[
            pltpu.VMEM((_B_PER_W,), jnp.int32),
            pltpu.VMEM((_B_PER_W, _HIDDEN), jnp.float32),
            pltpu.SemaphoreType.DMA,
        ],
    )
    def gather_kernel(idx_hbm, table_hbm, out_hbm, idx_v, rows_v, sem):
        wid = lax.axis_index("s") * _NC + lax.axis_index("c")
        base = wid * _B_PER_W
        pltpu.sync_copy(idx_hbm.at[pl.ds(base, _B_PER_W)], idx_v)
        pltpu.async_copy(table_hbm.at[idx_v], rows_v, sem).wait()
        pltpu.sync_copy(rows_v, out_hbm.at[pl.ds(base, _B_PER_W)])

    return gather_kernel(pert_ids, emb)


def _make_add_kernel(bb):
    def _add_kernel(x_ref, cond_ref, o_ref):
        i = pl.program_id(0)
        c = cond_ref[pl.ds(i * bb, bb), :]
        o_ref[...] = x_ref[...] + c[:, None, :]
    return _add_kernel


def _tc_broadcast_add(x, cond, bb=128, sq=40):
    def _add_kernel(x_ref, cond_ref, o_ref):
        i = pl.program_id(0)
        c = cond_ref[pl.ds(i * bb, bb), :]
        o_ref[...] = x_ref[...] + c[:, None, :]

    return pl.pallas_call(
        _add_kernel,
        grid=(_BATCH // bb, _SEQ // sq),
        in_specs=[
            pl.BlockSpec((bb, sq, _HIDDEN), lambda i, j: (i, j, 0)),
            pl.BlockSpec((_BATCH, _HIDDEN), lambda i, j: (0, 0)),
        ],
        out_specs=pl.BlockSpec((bb, sq, _HIDDEN), lambda i, j: (i, j, 0)),
        out_shape=jax.ShapeDtypeStruct((_BATCH, _SEQ, _HIDDEN), jnp.float32),
        compiler_params=pltpu.CompilerParams(
            dimension_semantics=("parallel", "arbitrary"),
        ),
    )(x, cond)


def kernel(x, pert_ids, emb):
    cond = _sc_gather(pert_ids.astype(jnp.int32), emb)
    return _tc_broadcast_add(x, cond)


# speedup vs baseline: 1.0664x; 1.0664x over previous
import functools

import jax
import jax.numpy as jnp
from jax import lax
from jax.experimental import pallas as pl
from jax.experimental.pallas import tpu as pltpu
from jax.experimental.pallas import tpu_sc as plsc

_BATCH = 1024
_SEQ = 200
_HIDDEN = 128

_info = plsc.get_sparse_core_info()
_NC = _info.num_cores
_NS = _info.num_subcores
_NW = _NC * _NS
_B_PER_W = _BATCH // _NW


def _sc_gather(pert_ids, emb):
    mesh = plsc.VectorSubcoreMesh(core_axis_name="c", subcore_axis_name="s")

    @functools.partial(
        pl.kernel,
        mesh=mesh,
        out_type=jax.ShapeDtypeStruct((_BATCH, _HIDDEN), jnp.float32),
        scratch_types=[
            pltpu.VMEM((_B_PER_W,), jnp.int32),
            pltpu.VMEM((_B_PER_W, _HIDDEN), jnp.float32),
            pltpu.SemaphoreType.DMA,
        ],
    )
    def gather_kernel(idx_hbm, table_hbm, out_hbm, idx_v, rows_v, sem):
        wid = lax.axis_index("s") * _NC + lax.axis_index("c")
        base = wid * _B_PER_W
        pltpu.sync_copy(idx_hbm.at[pl.ds(base, _B_PER_W)], idx_v)
        pltpu.async_copy(table_hbm.at[idx_v], rows_v, sem).wait()
        pltpu.sync_copy(rows_v, out_hbm.at[pl.ds(base, _B_PER_W)])

    return gather_kernel(pert_ids, emb)


def _make_add_kernel(bb):
    def _add_kernel(x_ref, cond_ref, o_ref):
        i = pl.program_id(0)
        c = cond_ref[pl.ds(i * bb, bb), :]
        o_ref[...] = x_ref[...] + c[:, None, :]
    return _add_kernel


def _tc_broadcast_add(x, cond, bb=128):
    def _add_kernel(x_ref, cond_ref, o_ref):
        i = pl.program_id(0)
        c = cond_ref[pl.ds(i * bb, bb), :]
        o_ref[...] = x_ref[...] + c[:, None, :]

    return pl.pallas_call(
        _add_kernel,
        grid=(_BATCH // bb,),
        in_specs=[
            pl.BlockSpec((bb, _SEQ, _HIDDEN), lambda i: (i, 0, 0)),
            pl.BlockSpec((_BATCH, _HIDDEN), lambda i: (0, 0)),
        ],
        out_specs=pl.BlockSpec((bb, _SEQ, _HIDDEN), lambda i: (i, 0, 0)),
        out_shape=jax.ShapeDtypeStruct((_BATCH, _SEQ, _HIDDEN), jnp.float32),
        compiler_params=pltpu.CompilerParams(
            dimension_semantics=("parallel",),
        ),
    )(x, cond)


def kernel(x, pert_ids, emb):
    cond = _sc_gather(pert_ids.astype(jnp.int32), emb)
    return _tc_broadcast_add(x, cond)
